# baseline (speedup 1.0000x reference)
import functools

import jax
import jax.numpy as jnp
from jax import lax
from jax.experimental import pallas as pl
from jax.experimental.pallas import tpu as pltpu
from jax.experimental.pallas import tpu_sc as plsc

N = 10000
E = 320000
D = 128

NC = 2
NS = 16
NW = NC * NS
E_PER_W = E // NW
CHUNK = 125
NCHUNK = E_PER_W // CHUNK
AGG_CHUNK = 50
AGG_NCHUNK = E_PER_W // AGG_CHUNK
WIN = 40
NWIN = AGG_NCHUNK // WIN
NBUF = 4
R_SLICE = 624
R_TAIL = N - NS * R_SLICE

_MESH = plsc.VectorSubcoreMesh(core_axis_name="c", subcore_axis_name="s")


@functools.partial(
    pl.kernel,
    out_type=jax.ShapeDtypeStruct((NC, N), jnp.float32),
    mesh=_MESH,
    scratch_types=[
        pltpu.VMEM((NCHUNK, CHUNK), jnp.int32),
        pltpu.VMEM((CHUNK,), jnp.float32),
        pltpu.VMEM_SHARED((N,), jnp.float32),
    ],
    name="sc_degree_hist",
)
def _degree_hist(dst_hbm, ones_hbm, zeros_hbm, cnt_hbm, dst_idx, ones_v, cnt):
    c = lax.axis_index("c")
    s = lax.axis_index("s")
    w = c * NS + s

    pltpu.sync_copy(dst_hbm.at[w], dst_idx)
    pltpu.sync_copy(ones_hbm, ones_v)

    @pl.when(s == 0)
    def _():
        pltpu.sync_copy(zeros_hbm, cnt)

    plsc.subcore_barrier()

    def body(j, carry):
        pltpu.sync_copy(ones_v, cnt.at[dst_idx.at[j]], add=True)
        return carry

    lax.fori_loop(0, NCHUNK, body, 0)
    plsc.subcore_barrier()

    @pl.when(s == 0)
    def _():
        pltpu.sync_copy(cnt, cnt_hbm.at[c])


@functools.partial(
    pl.kernel,
    out_type=jax.ShapeDtypeStruct((NC, N, D), jnp.float32),
    mesh=_MESH,
    scratch_types=[
        pltpu.VMEM((WIN, AGG_CHUNK), jnp.int32),
        pltpu.VMEM((WIN, AGG_CHUNK), jnp.int32),
        [pltpu.VMEM((AGG_CHUNK, D), jnp.float32) for _ in range(NBUF)],
        [pltpu.SemaphoreType.DMA for _ in range(NBUF)],
        [pltpu.SemaphoreType.DMA for _ in range(NBUF)],
        pltpu.VMEM_SHARED((N, D), jnp.float32),
    ],
    name="sc_edge_aggregate",
)
def _edge_aggregate(y_hbm, src_hbm, dst_hbm, zeros_hbm, acc_hbm,
                    src_idx, dst_idx, rows, gsem, ssem, acc):
    c = lax.axis_index("c")
    s = lax.axis_index("s")
    w = c * NS + s

    pltpu.sync_copy(zeros_hbm.at[pl.ds(0, R_SLICE)],
                    acc.at[pl.ds(s * R_SLICE, R_SLICE)])

    @pl.when(s == 0)
    def _():
        pltpu.sync_copy(zeros_hbm.at[pl.ds(0, R_TAIL)],
                        acc.at[pl.ds(NS * R_SLICE, R_TAIL)])

    plsc.subcore_barrier()

    def window(wi, carry):
        pltpu.sync_copy(src_hbm.at[w, pl.ds(wi * WIN, WIN)], src_idx)
        pltpu.sync_copy(dst_hbm.at[w, pl.ds(wi * WIN, WIN)], dst_idx)
        pltpu.async_copy(y_hbm.at[src_idx.at[0]], rows[0], gsem[0])
        pltpu.async_copy(y_hbm.at[src_idx.at[1]], rows[1], gsem[1])

        def body(p, carry2):
            for q in range(NBUF):
                j = NBUF * p + q
                q2 = (q + 2) % NBUF
                pltpu.make_async_copy(y_hbm.at[src_idx.at[j]], rows[q],
                                      gsem[q]).wait()
                pltpu.async_copy(rows[q], acc.at[dst_idx.at[j]], ssem[q],
                                 add=True)

                @pl.when(j >= 2)
                def _():
                    pltpu.make_async_copy(rows[q2], acc.at[dst_idx.at[0]],
                                          ssem[q2]).wait()

                @pl.when(j + 2 < WIN)
                def _():
                    pltpu.async_copy(y_hbm.at[src_idx.at[j + 2]], rows[q2],
                                     gsem[q2])

            return carry2

        lax.fori_loop(0, WIN // NBUF, body, 0)
        pltpu.make_async_copy(rows[(WIN - 2) % NBUF], acc.at[dst_idx.at[0]],
                              ssem[(WIN - 2) % NBUF]).wait()
        pltpu.make_async_copy(rows[(WIN - 1) % NBUF], acc.at[dst_idx.at[0]],
                              ssem[(WIN - 1) % NBUF]).wait()
        return carry

    lax.fori_loop(0, NWIN, window, 0)
    plsc.subcore_barrier()
    pltpu.sync_copy(
        acc.at[pl.ds(s * R_SLICE, R_SLICE)],
        acc_hbm.at[c, pl.ds(s * R_SLICE, R_SLICE)],
    )

    @pl.when(s == 0)
    def _():
        pltpu.sync_copy(
            acc.at[pl.ds(NS * R_SLICE, R_TAIL)],
            acc_hbm.at[c, pl.ds(NS * R_SLICE, R_TAIL)],
        )


_BLK = 1000


def _mm_body(x_ref, w_ref, cnt_ref, y_ref, dis_ref):
    xw = jnp.dot(x_ref[...], w_ref[...], preferred_element_type=jnp.float32)
    deg = 1.0 + cnt_ref[:, 0:1] + cnt_ref[:, 1:2]
    dis = lax.rsqrt(deg)
    y_ref[...] = xw * dis
    dis_ref[...] = dis


def _mm_scale(x, W, cntT):
    return pl.pallas_call(
        _mm_body,
        grid=(N // _BLK,),
        in_specs=[
            pl.BlockSpec((_BLK, D), lambda i: (i, 0)),
            pl.BlockSpec((D, D), lambda i: (0, 0)),
            pl.BlockSpec((_BLK, NC), lambda i: (i, 0)),
        ],
        out_specs=[
            pl.BlockSpec((_BLK, D), lambda i: (i, 0)),
            pl.BlockSpec((_BLK, 1), lambda i: (i, 0)),
        ],
        out_shape=[
            jax.ShapeDtypeStruct((N, D), jnp.float32),
            jax.ShapeDtypeStruct((N, 1), jnp.float32),
        ],
    )(x, W, cntT)


def _fin_body(acc_ref, y_ref, dis_ref, b_ref, pw_ref, o_ref):
    t = acc_ref[0] + acc_ref[1] + y_ref[...]
    t = t * dis_ref[...] + b_ref[...]
    t = jnp.where(t >= 0, t, pw_ref[...] * t)
    nrm = jnp.sqrt(jnp.sum(t * t, axis=1, keepdims=True))
    o_ref[...] = t / jnp.maximum(nrm, 1e-12)


def _finalize(acc, y, dis, b2, pw2):
    return pl.pallas_call(
        _fin_body,
        grid=(N // _BLK,),
        in_specs=[
            pl.BlockSpec((NC, _BLK, D), lambda i: (0, i, 0)),
            pl.BlockSpec((_BLK, D), lambda i: (i, 0)),
            pl.BlockSpec((_BLK, 1), lambda i: (i, 0)),
            pl.BlockSpec((1, D), lambda i: (0, 0)),
            pl.BlockSpec((1, D), lambda i: (0, 0)),
        ],
        out_specs=pl.BlockSpec((_BLK, D), lambda i: (i, 0)),
        out_shape=jax.ShapeDtypeStruct((N, D), jnp.float32),
    )(acc, y, dis, b2, pw2)


def kernel(x, edge_index, W, b, prelu_w):
    ei = edge_index.astype(jnp.int32)
    src_r = ei[0].reshape(NW, AGG_NCHUNK, AGG_CHUNK)
    dst_r = ei[1].reshape(NW, AGG_NCHUNK, AGG_CHUNK)
    dst_h = ei[1].reshape(NW, NCHUNK, CHUNK)

    ones_c = jnp.ones((CHUNK,), jnp.float32)
    zeros_n = jnp.zeros((N,), jnp.float32)
    zeros_rows = jnp.zeros((R_SLICE, D), jnp.float32)

    cnt = _degree_hist(dst_h, ones_c, zeros_n)
    y, dis = _mm_scale(x, W, cnt.T)
    acc = _edge_aggregate(y, src_r, dst_r, zeros_rows)
    return _finalize(acc, y, dis, b[None, :], prelu_w[None, :])


# speedup vs baseline: 1.0906x; 1.0906x over previous
import functools

import jax
import jax.numpy as jnp
from jax import lax
from jax.experimental import pallas as pl
from jax.experimental.pallas import tpu as pltpu
from jax.experimental.pallas import tpu_sc as plsc

N = 10000
E = 320000
D = 128

NC = 2
NS = 16
NW = NC * NS
E_PER_W = E // NW
CHUNK = 125
NCHUNK = E_PER_W // CHUNK
WIN = 16
NWIN = NCHUNK // WIN
NBUF = 2
R_SLICE = 624
R_TAIL = N - NS * R_SLICE

_MESH = plsc.VectorSubcoreMesh(core_axis_name="c", subcore_axis_name="s")


@functools.partial(
    pl.kernel,
    out_type=jax.ShapeDtypeStruct((NC, N), jnp.float32),
    mesh=_MESH,
    scratch_types=[
        pltpu.VMEM((NCHUNK, CHUNK), jnp.int32),
        pltpu.VMEM((CHUNK,), jnp.float32),
        pltpu.VMEM_SHARED((N,), jnp.float32),
    ],
    name="sc_degree_hist",
)
def _degree_hist(dst_hbm, ones_hbm, zeros_hbm, cnt_hbm, dst_idx, ones_v, cnt):
    c = lax.axis_index("c")
    s = lax.axis_index("s")
    w = c * NS + s

    pltpu.sync_copy(dst_hbm.at[w], dst_idx)
    pltpu.sync_copy(ones_hbm, ones_v)

    @pl.when(s == 0)
    def _():
        pltpu.sync_copy(zeros_hbm, cnt)

    plsc.subcore_barrier()

    def body(j, carry):
        pltpu.sync_copy(ones_v, cnt.at[dst_idx.at[j]], add=True)
        return carry

    lax.fori_loop(0, NCHUNK, body, 0)
    plsc.subcore_barrier()

    @pl.when(s == 0)
    def _():
        pltpu.sync_copy(cnt, cnt_hbm.at[c])


@functools.partial(
    pl.kernel,
    out_type=jax.ShapeDtypeStruct((NC, N, D), jnp.float32),
    mesh=_MESH,
    scratch_types=[
        pltpu.VMEM((WIN, CHUNK), jnp.int32),
        pltpu.VMEM((WIN, CHUNK), jnp.int32),
        [pltpu.VMEM((CHUNK, D), jnp.float32) for _ in range(NBUF)],
        [pltpu.SemaphoreType.DMA for _ in range(NBUF)],
        [pltpu.SemaphoreType.DMA for _ in range(NBUF)],
        pltpu.VMEM_SHARED((N, D), jnp.float32),
    ],
    name="sc_edge_aggregate",
)
def _edge_aggregate(y_hbm, src_hbm, dst_hbm, zeros_hbm, acc_hbm,
                    src_idx, dst_idx, rows, gsem, ssem, acc):
    c = lax.axis_index("c")
    s = lax.axis_index("s")
    w = c * NS + s

    pltpu.sync_copy(zeros_hbm.at[pl.ds(0, R_SLICE)],
                    acc.at[pl.ds(s * R_SLICE, R_SLICE)])

    @pl.when(s == 0)
    def _():
        pltpu.sync_copy(zeros_hbm.at[pl.ds(0, R_TAIL)],
                        acc.at[pl.ds(NS * R_SLICE, R_TAIL)])

    plsc.subcore_barrier()

    def window(wi, carry):
        pltpu.sync_copy(src_hbm.at[w, pl.ds(wi * WIN, WIN)], src_idx)
        pltpu.sync_copy(dst_hbm.at[w, pl.ds(wi * WIN, WIN)], dst_idx)
        pltpu.async_copy(y_hbm.at[src_idx.at[0]], rows[0], gsem[0])

        def body(p, carry2):
            for q in range(NBUF):
                j = NBUF * p + q
                qo = q ^ 1

                @pl.when((j >= 1) & (j <= WIN - 2))
                def _():
                    pltpu.make_async_copy(rows[qo], acc.at[dst_idx.at[0]],
                                          ssem[qo]).wait()
                    pltpu.async_copy(y_hbm.at[src_idx.at[j + 1]], rows[qo],
                                     gsem[qo])

                @pl.when(j == 0)
                def _():
                    pltpu.async_copy(y_hbm.at[src_idx.at[1]], rows[1], gsem[1])

                pltpu.make_async_copy(y_hbm.at[src_idx.at[j]], rows[q],
                                      gsem[q]).wait()
                pltpu.async_copy(rows[q], acc.at[dst_idx.at[j]], ssem[q],
                                 add=True)
            return carry2

        lax.fori_loop(0, WIN // NBUF, body, 0)
        pltpu.make_async_copy(rows[0], acc.at[dst_idx.at[0]], ssem[0]).wait()
        pltpu.make_async_copy(rows[1], acc.at[dst_idx.at[0]], ssem[1]).wait()
        return carry

    lax.fori_loop(0, NWIN, window, 0)
    plsc.subcore_barrier()
    pltpu.sync_copy(
        acc.at[pl.ds(s * R_SLICE, R_SLICE)],
        acc_hbm.at[c, pl.ds(s * R_SLICE, R_SLICE)],
    )

    @pl.when(s == 0)
    def _():
        pltpu.sync_copy(
            acc.at[pl.ds(NS * R_SLICE, R_TAIL)],
            acc_hbm.at[c, pl.ds(NS * R_SLICE, R_TAIL)],
        )


_BLK = 1000


def _mm_body(x_ref, w_ref, cnt_ref, y_ref, dis_ref):
    xw = jnp.dot(x_ref[...], w_ref[...], preferred_element_type=jnp.float32)
    deg = 1.0 + cnt_ref[:, 0:1] + cnt_ref[:, 1:2]
    dis = lax.rsqrt(deg)
    y_ref[...] = xw * dis
    dis_ref[...] = dis


def _mm_scale(x, W, cntT):
    return pl.pallas_call(
        _mm_body,
        grid=(N // _BLK,),
        in_specs=[
            pl.BlockSpec((_BLK, D), lambda i: (i, 0)),
            pl.BlockSpec((D, D), lambda i: (0, 0)),
            pl.BlockSpec((_BLK, NC), lambda i: (i, 0)),
        ],
        out_specs=[
            pl.BlockSpec((_BLK, D), lambda i: (i, 0)),
            pl.BlockSpec((_BLK, 1), lambda i: (i, 0)),
        ],
        out_shape=[
            jax.ShapeDtypeStruct((N, D), jnp.float32),
            jax.ShapeDtypeStruct((N, 1), jnp.float32),
        ],
    )(x, W, cntT)


def _fin_body(acc_ref, y_ref, dis_ref, b_ref, pw_ref, o_ref):
    t = acc_ref[0] + acc_ref[1] + y_ref[...]
    t = t * dis_ref[...] + b_ref[...]
    t = jnp.where(t >= 0, t, pw_ref[...] * t)
    nrm = jnp.sqrt(jnp.sum(t * t, axis=1, keepdims=True))
    o_ref[...] = t / jnp.maximum(nrm, 1e-12)


def _finalize(acc, y, dis, b2, pw2):
    return pl.pallas_call(
        _fin_body,
        grid=(N // _BLK,),
        in_specs=[
            pl.BlockSpec((NC, _BLK, D), lambda i: (0, i, 0)),
            pl.BlockSpec((_BLK, D), lambda i: (i, 0)),
            pl.BlockSpec((_BLK, 1), lambda i: (i, 0)),
            pl.BlockSpec((1, D), lambda i: (0, 0)),
            pl.BlockSpec((1, D), lambda i: (0, 0)),
        ],
        out_specs=pl.BlockSpec((_BLK, D), lambda i: (i, 0)),
        out_shape=jax.ShapeDtypeStruct((N, D), jnp.float32),
    )(acc, y, dis, b2, pw2)


def kernel(x, edge_index, W, b, prelu_w):
    ei = edge_index.astype(jnp.int32)
    src_r = ei[0].reshape(NW, NCHUNK, CHUNK)
    dst_r = ei[1].reshape(NW, NCHUNK, CHUNK)

    ones_c = jnp.ones((CHUNK,), jnp.float32)
    zeros_n = jnp.zeros((N,), jnp.float32)
    zeros_rows = jnp.zeros((R_SLICE, D), jnp.float32)

    cnt = _degree_hist(dst_r, ones_c, zeros_n)
    y, dis = _mm_scale(x, W, cnt.T)
    acc = _edge_aggregate(y, src_r, dst_r, zeros_rows)
    return _finalize(acc, y, dis, b[None, :], prelu_w[None, :])


# speedup vs baseline: 1.1374x; 1.0429x over previous
import functools

import jax
import jax.numpy as jnp
from jax import lax
from jax.experimental import pallas as pl
from jax.experimental.pallas import tpu as pltpu
from jax.experimental.pallas import tpu_sc as plsc

N = 10000
E = 320000
D = 128

NC = 2
NS = 16
NW = NC * NS
E_PER_W = E // NW
CHUNK = 125
NCHUNK = E_PER_W // CHUNK
WIN = 16
NWIN = NCHUNK // WIN
NBUF = 2
R_SLICE = 624
R_TAIL = N - NS * R_SLICE

_MESH = plsc.VectorSubcoreMesh(core_axis_name="c", subcore_axis_name="s")


@functools.partial(
    pl.kernel,
    out_type=jax.ShapeDtypeStruct((NC, N), jnp.float32),
    mesh=_MESH,
    scratch_types=[
        pltpu.VMEM((NCHUNK, CHUNK), jnp.int32),
        pltpu.VMEM((CHUNK,), jnp.float32),
        pltpu.VMEM_SHARED((N,), jnp.float32),
    ],
    name="sc_degree_hist",
)
def _degree_hist(edges_hbm, ones_hbm, zeros_hbm, cnt_hbm, dst_idx, ones_v, cnt):
    c = lax.axis_index("c")
    s = lax.axis_index("s")
    w = c * NS + s

    pltpu.sync_copy(edges_hbm.at[1, w], dst_idx)
    pltpu.sync_copy(ones_hbm, ones_v)

    @pl.when(s == 0)
    def _():
        pltpu.sync_copy(zeros_hbm, cnt)

    plsc.subcore_barrier()

    def body(j, carry):
        pltpu.sync_copy(ones_v, cnt.at[dst_idx.at[j]], add=True)
        return carry

    lax.fori_loop(0, NCHUNK, body, 0)
    plsc.subcore_barrier()

    @pl.when(s == 0)
    def _():
        pltpu.sync_copy(cnt, cnt_hbm.at[c])


@functools.partial(
    pl.kernel,
    out_type=jax.ShapeDtypeStruct((NC, N, D), jnp.float32),
    mesh=_MESH,
    scratch_types=[
        pltpu.VMEM((WIN, CHUNK), jnp.int32),
        pltpu.VMEM((WIN, CHUNK), jnp.int32),
        [pltpu.VMEM((CHUNK, D), jnp.float32) for _ in range(NBUF)],
        [pltpu.SemaphoreType.DMA for _ in range(NBUF)],
        [pltpu.SemaphoreType.DMA for _ in range(NBUF)],
        pltpu.VMEM_SHARED((N, D), jnp.float32),
    ],
    name="sc_edge_aggregate",
)
def _edge_aggregate(y_hbm, edges_hbm, zeros_hbm, acc_hbm,
                    src_idx, dst_idx, rows, gsem, ssem, acc):
    c = lax.axis_index("c")
    s = lax.axis_index("s")
    w = c * NS + s

    pltpu.sync_copy(zeros_hbm.at[pl.ds(0, R_SLICE)],
                    acc.at[pl.ds(s * R_SLICE, R_SLICE)])

    @pl.when(s == 0)
    def _():
        pltpu.sync_copy(zeros_hbm.at[pl.ds(0, R_TAIL)],
                        acc.at[pl.ds(NS * R_SLICE, R_TAIL)])

    plsc.subcore_barrier()

    def window(wi, carry):
        pltpu.sync_copy(edges_hbm.at[0, w, pl.ds(wi * WIN, WIN)], src_idx)
        pltpu.sync_copy(edges_hbm.at[1, w, pl.ds(wi * WIN, WIN)], dst_idx)
        pltpu.async_copy(y_hbm.at[src_idx.at[0]], rows[0], gsem[0])

        def body(p, carry2):
            for q in range(NBUF):
                j = NBUF * p + q
                qo = q ^ 1

                @pl.when((j >= 1) & (j <= WIN - 2))
                def _():
                    pltpu.make_async_copy(rows[qo], acc.at[dst_idx.at[0]],
                                          ssem[qo]).wait()
                    pltpu.async_copy(y_hbm.at[src_idx.at[j + 1]], rows[qo],
                                     gsem[qo])

                @pl.when(j == 0)
                def _():
                    pltpu.async_copy(y_hbm.at[src_idx.at[1]], rows[1], gsem[1])

                pltpu.make_async_copy(y_hbm.at[src_idx.at[j]], rows[q],
                                      gsem[q]).wait()
                pltpu.async_copy(rows[q], acc.at[dst_idx.at[j]], ssem[q],
                                 add=True)
            return carry2

        lax.fori_loop(0, WIN // NBUF, body, 0)
        pltpu.make_async_copy(rows[0], acc.at[dst_idx.at[0]], ssem[0]).wait()
        pltpu.make_async_copy(rows[1], acc.at[dst_idx.at[0]], ssem[1]).wait()
        return carry

    lax.fori_loop(0, NWIN, window, 0)
    plsc.subcore_barrier()
    pltpu.sync_copy(
        acc.at[pl.ds(s * R_SLICE, R_SLICE)],
        acc_hbm.at[c, pl.ds(s * R_SLICE, R_SLICE)],
    )

    @pl.when(s == 0)
    def _():
        pltpu.sync_copy(
            acc.at[pl.ds(NS * R_SLICE, R_TAIL)],
            acc_hbm.at[c, pl.ds(NS * R_SLICE, R_TAIL)],
        )


_BLK = 1000


def _mm_body(x_ref, w_ref, xw_ref):
    xw_ref[...] = jnp.dot(x_ref[...], w_ref[...],
                          preferred_element_type=jnp.float32)


def _matmul(x, W):
    return pl.pallas_call(
        _mm_body,
        grid=(N // _BLK,),
        in_specs=[
            pl.BlockSpec((_BLK, D), lambda i: (i, 0)),
            pl.BlockSpec((D, D), lambda i: (0, 0)),
        ],
        out_specs=pl.BlockSpec((_BLK, D), lambda i: (i, 0)),
        out_shape=jax.ShapeDtypeStruct((N, D), jnp.float32),
    )(x, W)


def _scale_body(xw_ref, cnt_ref, y_ref, dis_ref):
    deg = 1.0 + cnt_ref[:, 0:1] + cnt_ref[:, 1:2]
    dis = lax.rsqrt(deg)
    y_ref[...] = xw_ref[...] * dis
    dis_ref[...] = dis


def _scale(xw, cntT):
    return pl.pallas_call(
        _scale_body,
        grid=(N // _BLK,),
        in_specs=[
            pl.BlockSpec((_BLK, D), lambda i: (i, 0)),
            pl.BlockSpec((_BLK, NC), lambda i: (i, 0)),
        ],
        out_specs=[
            pl.BlockSpec((_BLK, D), lambda i: (i, 0)),
            pl.BlockSpec((_BLK, 1), lambda i: (i, 0)),
        ],
        out_shape=[
            jax.ShapeDtypeStruct((N, D), jnp.float32),
            jax.ShapeDtypeStruct((N, 1), jnp.float32),
        ],
    )(xw, cntT)


def _fin_body(acc_ref, y_ref, dis_ref, b_ref, pw_ref, o_ref):
    t = acc_ref[0] + acc_ref[1] + y_ref[...]
    t = t * dis_ref[...] + b_ref[...]
    t = jnp.where(t >= 0, t, pw_ref[...] * t)
    nrm = jnp.sqrt(jnp.sum(t * t, axis=1, keepdims=True))
    o_ref[...] = t / jnp.maximum(nrm, 1e-12)


def _finalize(acc, y, dis, b2, pw2):
    return pl.pallas_call(
        _fin_body,
        grid=(N // _BLK,),
        in_specs=[
            pl.BlockSpec((NC, _BLK, D), lambda i: (0, i, 0)),
            pl.BlockSpec((_BLK, D), lambda i: (i, 0)),
            pl.BlockSpec((_BLK, 1), lambda i: (i, 0)),
            pl.BlockSpec((1, D), lambda i: (0, 0)),
            pl.BlockSpec((1, D), lambda i: (0, 0)),
        ],
        out_specs=pl.BlockSpec((_BLK, D), lambda i: (i, 0)),
        out_shape=jax.ShapeDtypeStruct((N, D), jnp.float32),
    )(acc, y, dis, b2, pw2)


def kernel(x, edge_index, W, b, prelu_w):
    if edge_index.dtype != jnp.int32:
        edge_index = edge_index.astype(jnp.int32)
    er = edge_index.reshape(2, NW, NCHUNK, CHUNK)

    ones_c = jnp.ones((CHUNK,), jnp.float32)
    zeros_n = jnp.zeros((N,), jnp.float32)
    zeros_rows = jnp.zeros((R_SLICE, D), jnp.float32)

    cnt = _degree_hist(er, ones_c, zeros_n)
    xw = _matmul(x, W)
    y, dis = _scale(xw, cnt.T)
    acc = _edge_aggregate(y, er, zeros_rows)
    return _finalize(acc, y, dis, b[None, :], prelu_w[None, :])


# speedup vs baseline: 1.1590x; 1.0190x over previous
import functools

import jax
import jax.numpy as jnp
from jax import lax
from jax.experimental import pallas as pl
from jax.experimental.pallas import tpu as pltpu
from jax.experimental.pallas import tpu_sc as plsc

N = 10000
E = 320000
D = 128

NC = 2
NS = 16
NW = NC * NS
E_PER_W = E // NW
CHUNK = 125
NCHUNK = E_PER_W // CHUNK
WIN = 16
NWIN = NCHUNK // WIN
NBUF = 2
R_SLICE = 624
R_TAIL = N - NS * R_SLICE

_MESH = plsc.VectorSubcoreMesh(core_axis_name="c", subcore_axis_name="s")


@functools.partial(
    pl.kernel,
    out_type=jax.ShapeDtypeStruct((NC, N), jnp.float32),
    mesh=_MESH,
    scratch_types=[
        pltpu.VMEM((NCHUNK, CHUNK), jnp.int32),
        pltpu.VMEM((CHUNK,), jnp.float32),
        pltpu.VMEM_SHARED((N,), jnp.float32),
    ],
    name="sc_degree_hist",
)
def _degree_hist(edges_hbm, ones_hbm, zeros_hbm, cnt_hbm, dst_idx, ones_v, cnt):
    c = lax.axis_index("c")
    s = lax.axis_index("s")
    w = c * NS + s

    pltpu.sync_copy(edges_hbm.at[1, w], dst_idx)
    pltpu.sync_copy(ones_hbm, ones_v)

    @pl.when(s == 0)
    def _():
        pltpu.sync_copy(zeros_hbm, cnt)

    plsc.subcore_barrier()

    def body(j, carry):
        pltpu.sync_copy(ones_v, cnt.at[dst_idx.at[j]], add=True)
        return carry

    lax.fori_loop(0, NCHUNK, body, 0)
    plsc.subcore_barrier()

    @pl.when(s == 0)
    def _():
        pltpu.sync_copy(cnt, cnt_hbm.at[c])


@functools.partial(
    pl.kernel,
    out_type=jax.ShapeDtypeStruct((NC, N, D), jnp.float32),
    mesh=_MESH,
    scratch_types=[
        pltpu.VMEM((WIN, CHUNK), jnp.int32),
        pltpu.VMEM((WIN, CHUNK), jnp.int32),
        [pltpu.VMEM((CHUNK, D), jnp.float32) for _ in range(NBUF)],
        [pltpu.SemaphoreType.DMA for _ in range(NBUF)],
        [pltpu.SemaphoreType.DMA for _ in range(NBUF)],
        pltpu.VMEM_SHARED((N, D), jnp.float32),
    ],
    name="sc_edge_aggregate",
)
def _edge_aggregate(y_hbm, edges_hbm, zeros_hbm, acc_hbm,
                    src_idx, dst_idx, rows, gsem, ssem, acc):
    c = lax.axis_index("c")
    s = lax.axis_index("s")
    w = c * NS + s

    pltpu.sync_copy(zeros_hbm.at[pl.ds(0, R_SLICE)],
                    acc.at[pl.ds(s * R_SLICE, R_SLICE)])

    @pl.when(s == 0)
    def _():
        pltpu.sync_copy(zeros_hbm.at[pl.ds(0, R_TAIL)],
                        acc.at[pl.ds(NS * R_SLICE, R_TAIL)])

    plsc.subcore_barrier()

    def window(wi, carry):
        pltpu.sync_copy(edges_hbm.at[0, w, pl.ds(wi * WIN, WIN)], src_idx)
        pltpu.sync_copy(edges_hbm.at[1, w, pl.ds(wi * WIN, WIN)], dst_idx)
        pltpu.async_copy(y_hbm.at[src_idx.at[0]], rows[0], gsem[0])

        def body(p, carry2):
            for q in range(NBUF):
                j = NBUF * p + q
                qo = q ^ 1

                @pl.when((j >= 1) & (j <= WIN - 2))
                def _():
                    pltpu.make_async_copy(rows[qo], acc.at[dst_idx.at[0]],
                                          ssem[qo]).wait()
                    pltpu.async_copy(y_hbm.at[src_idx.at[j + 1]], rows[qo],
                                     gsem[qo])

                @pl.when(j == 0)
                def _():
                    pltpu.async_copy(y_hbm.at[src_idx.at[1]], rows[1], gsem[1])

                pltpu.make_async_copy(y_hbm.at[src_idx.at[j]], rows[q],
                                      gsem[q]).wait()
                pltpu.async_copy(rows[q], acc.at[dst_idx.at[j]], ssem[q],
                                 add=True)
            return carry2

        lax.fori_loop(0, WIN // NBUF, body, 0)
        pltpu.make_async_copy(rows[0], acc.at[dst_idx.at[0]], ssem[0]).wait()
        pltpu.make_async_copy(rows[1], acc.at[dst_idx.at[0]], ssem[1]).wait()
        return carry

    lax.fori_loop(0, NWIN, window, 0)
    plsc.subcore_barrier()
    pltpu.sync_copy(
        acc.at[pl.ds(s * R_SLICE, R_SLICE)],
        acc_hbm.at[c, pl.ds(s * R_SLICE, R_SLICE)],
    )

    @pl.when(s == 0)
    def _():
        pltpu.sync_copy(
            acc.at[pl.ds(NS * R_SLICE, R_TAIL)],
            acc_hbm.at[c, pl.ds(NS * R_SLICE, R_TAIL)],
        )


_BLK = 2000


def _mm_body(x_ref, w_ref, xw_ref):
    xw_ref[...] = jnp.dot(x_ref[...], w_ref[...],
                          preferred_element_type=jnp.float32)


def _matmul(x, W):
    return pl.pallas_call(
        _mm_body,
        grid=(N // _BLK,),
        in_specs=[
            pl.BlockSpec((_BLK, D), lambda i: (i, 0)),
            pl.BlockSpec((D, D), lambda i: (0, 0)),
        ],
        out_specs=pl.BlockSpec((_BLK, D), lambda i: (i, 0)),
        out_shape=jax.ShapeDtypeStruct((N, D), jnp.float32),
    )(x, W)


def _scale_body(xw_ref, cnt_ref, y_ref, dis_ref):
    deg = 1.0 + cnt_ref[:, 0:1] + cnt_ref[:, 1:2]
    dis = lax.rsqrt(deg)
    y_ref[...] = xw_ref[...] * dis
    dis_ref[...] = dis


def _scale(xw, cntT):
    return pl.pallas_call(
        _scale_body,
        grid=(N // _BLK,),
        in_specs=[
            pl.BlockSpec((_BLK, D), lambda i: (i, 0)),
            pl.BlockSpec((_BLK, NC), lambda i: (i, 0)),
        ],
        out_specs=[
            pl.BlockSpec((_BLK, D), lambda i: (i, 0)),
            pl.BlockSpec((_BLK, 1), lambda i: (i, 0)),
        ],
        out_shape=[
            jax.ShapeDtypeStruct((N, D), jnp.float32),
            jax.ShapeDtypeStruct((N, 1), jnp.float32),
        ],
    )(xw, cntT)


def _fin_body(acc_ref, y_ref, dis_ref, b_ref, pw_ref, o_ref):
    t = acc_ref[0] + acc_ref[1] + y_ref[...]
    t = t * dis_ref[...] + b_ref[...]
    t = jnp.where(t >= 0, t, pw_ref[...] * t)
    nrm = jnp.sqrt(jnp.sum(t * t, axis=1, keepdims=True))
    o_ref[...] = t / jnp.maximum(nrm, 1e-12)


def _finalize(acc, y, dis, b2, pw2):
    return pl.pallas_call(
        _fin_body,
        grid=(N // _BLK,),
        in_specs=[
            pl.BlockSpec((NC, _BLK, D), lambda i: (0, i, 0)),
            pl.BlockSpec((_BLK, D), lambda i: (i, 0)),
            pl.BlockSpec((_BLK, 1), lambda i: (i, 0)),
            pl.BlockSpec((1, D), lambda i: (0, 0)),
            pl.BlockSpec((1, D), lambda i: (0, 0)),
        ],
        out_specs=pl.BlockSpec((_BLK, D), lambda i: (i, 0)),
        out_shape=jax.ShapeDtypeStruct((N, D), jnp.float32),
    )(acc, y, dis, b2, pw2)


def kernel(x, edge_index, W, b, prelu_w):
    if edge_index.dtype != jnp.int32:
        edge_index = edge_index.astype(jnp.int32)
    er = edge_index.reshape(2, NW, NCHUNK, CHUNK)

    ones_c = jnp.ones((CHUNK,), jnp.float32)
    zeros_n = jnp.zeros((N,), jnp.float32)
    zeros_rows = jnp.zeros((R_SLICE, D), jnp.float32)

    cnt = _degree_hist(er, ones_c, zeros_n)
    xw = _matmul(x, W)
    y, dis = _scale(xw, cnt.T)
    acc = _edge_aggregate(y, er, zeros_rows)
    return _finalize(acc, y, dis, b[None, :], prelu_w[None, :])
